# baseline (speedup 1.0000x reference)
import functools

import jax
import jax.numpy as jnp
from jax import lax
from jax.experimental import pallas as pl
from jax.experimental.pallas import tpu as pltpu
from jax.experimental.pallas import tpu_sc as plsc

B = 16384
EMB = 32
E = 8
PACK = 4
ROW = EMB * PACK

_CHUNK = 128
_CBLK = 16384
_QBLK = _CBLK // PACK
_N_USERS = 1000000
_N_BLK = -(-_N_USERS // _CBLK)
_PROWS = _N_BLK * _QBLK


def _repack_kernel(in_ref, out_ref):
    blk = in_ref[...]
    out_ref[0] = jnp.transpose(jnp.reshape(blk, (ROW, _QBLK)), (1, 0))


def _repack(table_t):
    out = pl.pallas_call(
        _repack_kernel,
        grid=(_N_BLK,),
        in_specs=[pl.BlockSpec((EMB, _CBLK), lambda i: (0, i))],
        out_specs=pl.BlockSpec((1, _QBLK, ROW), lambda i: (i, 0, 0)),
        out_shape=jax.ShapeDtypeStruct((_N_BLK, _QBLK, ROW), jnp.float32),
    )(table_t)
    return out.reshape(_PROWS, ROW)


def _make_sc_gather(num_rows):
    info = plsc.get_sparse_core_info()
    nw = info.num_cores * info.num_subcores
    b_per_w = num_rows // nw
    n_chunks = b_per_w // _CHUNK
    mesh = plsc.VectorSubcoreMesh(core_axis_name="c", subcore_axis_name="s")

    @functools.partial(
        pl.kernel,
        mesh=mesh,
        out_type=jax.ShapeDtypeStruct((num_rows, ROW), jnp.float32),
        scratch_types=[
            pltpu.VMEM((n_chunks, _CHUNK), jnp.int32),
            pltpu.VMEM((2, _CHUNK, ROW), jnp.float32),
            pltpu.SemaphoreType.DMA,
            pltpu.SemaphoreType.DMA,
        ],
    )
    def gather_kernel(uidx_hbm, wu_hbm, u_out, uidx_v, u_buf, gsem, wsem):
        wid = lax.axis_index("s") * info.num_cores + lax.axis_index("c")
        base = wid * b_per_w
        pltpu.sync_copy(uidx_hbm.at[pl.ds(wid * n_chunks, n_chunks)], uidx_v)
        writes = []
        for j in range(n_chunks):
            bb = j % 2
            if j >= 2:
                writes[j - 2].wait()
            cu = pltpu.async_copy(wu_hbm.at[uidx_v.at[j]], u_buf.at[bb], gsem)
            cu.wait()
            dst = pl.ds(base + j * _CHUNK, _CHUNK)
            writes.append(pltpu.async_copy(u_buf.at[bb], u_out.at[dst], wsem))
        for w in writes[-2:]:
            w.wait()

    return gather_kernel


def _head_kernel(u_ref, v_ref, us_ref, vs_ref, g_ref, lwu4_ref, lwv4_ref,
                 linb_ref, selw4_ref, selb_ref, a_ref, b_ref, t_ref, out_ref):
    u = u_ref[...]
    v = v_ref[...]
    pos = lax.broadcasted_iota(jnp.int32, (1, PACK), 1)
    ohu = (us_ref[...][:, None] == pos).astype(jnp.float32)
    ohv = (vs_ref[...][:, None] == pos).astype(jnp.float32)
    lu = jnp.dot(u, lwu4_ref[...], preferred_element_type=jnp.float32)
    lv = jnp.dot(v, lwv4_ref[...], preferred_element_type=jnp.float32)
    logit = (jnp.sum(ohu * lu, axis=1, keepdims=True)
             + jnp.sum(ohv * lv, axis=1, keepdims=True)
             + linb_ref[0, 0])
    s4 = jnp.dot(u, selw4_ref[...], preferred_element_type=jnp.float32)
    s = selb_ref[...]
    for p in range(PACK):
        s = s + ohu[:, p:p + 1] * s4[:, p * E:(p + 1) * E]
    s = s - jnp.max(s, axis=1, keepdims=True)
    es = jnp.exp(s)
    sd = es / jnp.sum(es, axis=1, keepdims=True) + 1e-10
    t = (jnp.log(sd) + g_ref[...]) / t_ref[0, 0]
    t = t - jnp.max(t, axis=1, keepdims=True)
    et = jnp.exp(t)
    w = et / jnp.sum(et, axis=1, keepdims=True)
    eo = 1.0 / (1.0 + jnp.exp(-(logit * a_ref[...] + b_ref[...])))
    r = jnp.sum(eo * w, axis=1)
    out_ref[...] = jnp.clip(r, 0.0, 1.0)


def _run_head(u_emb, v_emb, u_sub, v_sub, g, lwu4, lwv4, lin_b, selw4, sel_b,
              a_prop, b_prop, t):
    n_blk = 8
    rows = B // n_blk
    full = lambda s: pl.BlockSpec(s, lambda i: (0,) * len(s))
    out = pl.pallas_call(
        _head_kernel,
        grid=(n_blk,),
        in_specs=[
            pl.BlockSpec((rows, ROW), lambda i: (i, 0)),
            pl.BlockSpec((rows, ROW), lambda i: (i, 0)),
            pl.BlockSpec((rows,), lambda i: (i,)),
            pl.BlockSpec((rows,), lambda i: (i,)),
            pl.BlockSpec((rows, E), lambda i: (i, 0)),
            full((ROW, PACK)),
            full((ROW, PACK)),
            full((1, 1)),
            full((ROW, PACK * E)),
            full((1, E)),
            full((1, E)),
            full((1, E)),
            full((1, 1)),
        ],
        out_specs=pl.BlockSpec((rows,), lambda i: (i,)),
        out_shape=jax.ShapeDtypeStruct((B,), jnp.float32),
    )(u_emb, v_emb, u_sub, v_sub, g, lwu4, lwv4, lin_b.reshape(1, 1),
      selw4, sel_b.reshape(1, E), a_prop.reshape(1, E), b_prop.reshape(1, E),
      t)
    return out


def kernel(x, T, W_user, H_item, lin_w, lin_b, sel_w, sel_b, a_prop, b_prop, g):
    user_idx = x[:, 0]
    item_idx = x[:, 1]
    uq = ((user_idx // _CBLK) * _QBLK
          + user_idx % _QBLK).reshape(B // _CHUNK, _CHUNK)
    iq = ((item_idx // _CBLK) * _QBLK
          + item_idx % _QBLK).reshape(B // _CHUNK, _CHUNK)
    usub = (user_idx // _QBLK) % PACK
    isub = (item_idx // _QBLK) % PACK
    gather = _make_sc_gather(B)
    w4 = _repack(W_user.T)
    u_emb = gather(uq, w4)
    h4 = _repack(H_item.T)
    v_emb = gather(iq, h4)
    eye4 = jnp.eye(PACK, dtype=jnp.float32)
    lwu4 = (lin_w[:EMB][:, None] * eye4[None, :, :]).reshape(ROW, PACK)
    lwv4 = (lin_w[EMB:][:, None] * eye4[None, :, :]).reshape(ROW, PACK)
    selw4 = (sel_w[:, None, None, :] * eye4[None, :, :, None]).reshape(
        ROW, PACK * E)
    t = jnp.asarray(T, jnp.float32).reshape(1, 1)
    return _run_head(u_emb, v_emb, usub, isub, g, lwu4, lwv4, lin_b,
                     selw4, sel_b, a_prop, b_prop, t)


# speedup vs baseline: 1.0598x; 1.0598x over previous
import functools

import jax
import jax.numpy as jnp
from jax import lax
from jax.experimental import pallas as pl
from jax.experimental.pallas import tpu as pltpu
from jax.experimental.pallas import tpu_sc as plsc

B = 16384
EMB = 32
E = 8
PACK = 4
ROW = EMB * PACK

_CHUNK = 128
_CBLK = 16384
_QBLK = _CBLK // PACK
_N_USERS = 1000000
_N_BLK = -(-_N_USERS // _CBLK)
_PROWS = _N_BLK * _QBLK


_HROWS = _QBLK // 2


def _repack_kernel(in_ref, out_ref):
    blk = in_ref[...]
    t = jnp.transpose(jnp.reshape(blk, (ROW, _QBLK)), (1, 0))
    bi = jax.lax.bitcast_convert_type(t[:_HROWS], jnp.int32)
    bl = jax.lax.bitcast_convert_type(t[_HROWS:], jnp.int32)
    packed = (bi & jnp.int32(-65536)) | jax.lax.shift_right_logical(bl, 16)
    out_ref[0] = jax.lax.bitcast_convert_type(packed, jnp.float32)


def _repack(table_t):
    out = pl.pallas_call(
        _repack_kernel,
        grid=(_N_BLK,),
        in_specs=[pl.BlockSpec((EMB, _CBLK), lambda i: (0, i))],
        out_specs=pl.BlockSpec((1, _HROWS, ROW), lambda i: (i, 0, 0)),
        out_shape=jax.ShapeDtypeStruct((_N_BLK, _HROWS, ROW), jnp.float32),
    )(table_t)
    return out.reshape(_N_BLK * _HROWS, ROW)


def _make_sc_gather(num_rows):
    info = plsc.get_sparse_core_info()
    nw = info.num_cores * info.num_subcores
    b_per_w = num_rows // nw
    n_chunks = b_per_w // _CHUNK
    mesh = plsc.VectorSubcoreMesh(core_axis_name="c", subcore_axis_name="s")

    @functools.partial(
        pl.kernel,
        mesh=mesh,
        out_type=jax.ShapeDtypeStruct((num_rows, ROW), jnp.float32),
        scratch_types=[
            pltpu.VMEM((n_chunks, _CHUNK), jnp.int32),
            pltpu.VMEM((2, _CHUNK, ROW), jnp.float32),
            pltpu.SemaphoreType.DMA,
            pltpu.SemaphoreType.DMA,
        ],
    )
    def gather_kernel(uidx_hbm, wu_hbm, u_out, uidx_v, u_buf, gsem, wsem):
        wid = lax.axis_index("s") * info.num_cores + lax.axis_index("c")
        base = wid * b_per_w
        pltpu.sync_copy(uidx_hbm.at[pl.ds(wid * n_chunks, n_chunks)], uidx_v)
        writes = []
        for j in range(n_chunks):
            bb = j % 2
            if j >= 2:
                writes[j - 2].wait()
            cu = pltpu.async_copy(wu_hbm.at[uidx_v.at[j]], u_buf.at[bb], gsem)
            cu.wait()
            dst = pl.ds(base + j * _CHUNK, _CHUNK)
            writes.append(pltpu.async_copy(u_buf.at[bb], u_out.at[dst], wsem))
        for w in writes[-2:]:
            w.wait()

    return gather_kernel


def _unpack_half(x, half):
    bits = jax.lax.bitcast_convert_type(x, jnp.int32)
    hi = jax.lax.bitcast_convert_type(bits & jnp.int32(-65536), jnp.float32)
    lo = jax.lax.bitcast_convert_type(
        jax.lax.shift_left(bits, 16), jnp.float32)
    return jnp.where(half[:, None] == 0, hi, lo)


def _head_kernel(u_ref, v_ref, us_ref, vs_ref, uh_ref, vh_ref, g_ref,
                 lwu4_ref, lwv4_ref, linb_ref, selw4_ref, selb_ref, a_ref,
                 b_ref, t_ref, out_ref):
    u = _unpack_half(u_ref[...], uh_ref[...])
    v = _unpack_half(v_ref[...], vh_ref[...])
    lane_q = lax.broadcasted_iota(jnp.int32, (1, ROW), 1) % PACK
    u = jnp.where(lane_q == us_ref[...][:, None], u, 0.0)
    v = jnp.where(lane_q == vs_ref[...][:, None], v, 0.0)
    pos = lax.broadcasted_iota(jnp.int32, (1, PACK), 1)
    ohu = (us_ref[...][:, None] == pos).astype(jnp.float32)
    ohv = (vs_ref[...][:, None] == pos).astype(jnp.float32)
    lu = jnp.dot(u, lwu4_ref[...], preferred_element_type=jnp.float32)
    lv = jnp.dot(v, lwv4_ref[...], preferred_element_type=jnp.float32)
    logit = (jnp.sum(ohu * lu, axis=1, keepdims=True)
             + jnp.sum(ohv * lv, axis=1, keepdims=True)
             + linb_ref[0, 0])
    s4 = jnp.dot(u, selw4_ref[...], preferred_element_type=jnp.float32)
    s = selb_ref[...]
    for p in range(PACK):
        s = s + ohu[:, p:p + 1] * s4[:, p * E:(p + 1) * E]
    s = s - jnp.max(s, axis=1, keepdims=True)
    es = jnp.exp(s)
    sd = es / jnp.sum(es, axis=1, keepdims=True) + 1e-10
    t = (jnp.log(sd) + g_ref[...]) / t_ref[0, 0]
    t = t - jnp.max(t, axis=1, keepdims=True)
    et = jnp.exp(t)
    w = et / jnp.sum(et, axis=1, keepdims=True)
    eo = 1.0 / (1.0 + jnp.exp(-(logit * a_ref[...] + b_ref[...])))
    r = jnp.sum(eo * w, axis=1)
    out_ref[...] = jnp.clip(r, 0.0, 1.0)


def _run_head(u_emb, v_emb, u_sub, v_sub, u_half, v_half, g, lwu4, lwv4,
              lin_b, selw4, sel_b, a_prop, b_prop, t):
    n_blk = 8
    rows = B // n_blk
    full = lambda s: pl.BlockSpec(s, lambda i: (0,) * len(s))
    out = pl.pallas_call(
        _head_kernel,
        grid=(n_blk,),
        in_specs=[
            pl.BlockSpec((rows, ROW), lambda i: (i, 0)),
            pl.BlockSpec((rows, ROW), lambda i: (i, 0)),
            pl.BlockSpec((rows,), lambda i: (i,)),
            pl.BlockSpec((rows,), lambda i: (i,)),
            pl.BlockSpec((rows,), lambda i: (i,)),
            pl.BlockSpec((rows,), lambda i: (i,)),
            pl.BlockSpec((rows, E), lambda i: (i, 0)),
            full((ROW, PACK)),
            full((ROW, PACK)),
            full((1, 1)),
            full((ROW, PACK * E)),
            full((1, E)),
            full((1, E)),
            full((1, E)),
            full((1, 1)),
        ],
        out_specs=pl.BlockSpec((rows,), lambda i: (i,)),
        out_shape=jax.ShapeDtypeStruct((B,), jnp.float32),
    )(u_emb, v_emb, u_sub, v_sub, u_half, v_half, g, lwu4, lwv4,
      lin_b.reshape(1, 1), selw4, sel_b.reshape(1, E), a_prop.reshape(1, E),
      b_prop.reshape(1, E), t)
    return out


def kernel(x, T, W_user, H_item, lin_w, lin_b, sel_w, sel_b, a_prop, b_prop, g):
    user_idx = x[:, 0]
    item_idx = x[:, 1]
    uq = ((user_idx // _CBLK) * _HROWS
          + user_idx % _HROWS).reshape(B // _CHUNK, _CHUNK)
    iq = ((item_idx // _CBLK) * _HROWS
          + item_idx % _HROWS).reshape(B // _CHUNK, _CHUNK)
    usub = (user_idx // _QBLK) % PACK
    isub = (item_idx // _QBLK) % PACK
    uhalf = (user_idx // _HROWS) % 2
    ihalf = (item_idx // _HROWS) % 2
    gather = _make_sc_gather(B)
    w4 = _repack(W_user.T)
    u_emb = gather(uq, w4)
    h4 = _repack(H_item.T)
    v_emb = gather(iq, h4)
    eye4 = jnp.eye(PACK, dtype=jnp.float32)
    lwu4 = (lin_w[:EMB][:, None] * eye4[None, :, :]).reshape(ROW, PACK)
    lwv4 = (lin_w[EMB:][:, None] * eye4[None, :, :]).reshape(ROW, PACK)
    selw4 = (sel_w[:, None, None, :] * eye4[None, :, :, None]).reshape(
        ROW, PACK * E)
    t = jnp.asarray(T, jnp.float32).reshape(1, 1)
    return _run_head(u_emb, v_emb, usub, isub, uhalf, ihalf, g, lwu4, lwv4,
                     lin_b, selw4, sel_b, a_prop, b_prop, t)


# speedup vs baseline: 1.1006x; 1.0385x over previous
import functools

import jax
import jax.numpy as jnp
from jax import lax
from jax.experimental import pallas as pl
from jax.experimental.pallas import tpu as pltpu
from jax.experimental.pallas import tpu_sc as plsc

B = 16384
EMB = 32
E = 8
PACK = 4
ROW = EMB * PACK

_CHUNK = 128
_CBLK = 16384
_QBLK = _CBLK // PACK
_N_USERS = 1000000
_N_BLK = -(-_N_USERS // _CBLK)
_PROWS = _N_BLK * _QBLK


_HROWS = _QBLK // 2


def _repack_kernel(in_ref, out_ref):
    blk = in_ref[...]
    t = jnp.transpose(jnp.reshape(blk, (ROW, _QBLK)), (1, 0))
    bi = jax.lax.bitcast_convert_type(t[:_HROWS], jnp.int32)
    bl = jax.lax.bitcast_convert_type(t[_HROWS:], jnp.int32)
    packed = (bi & jnp.int32(-65536)) | jax.lax.shift_right_logical(bl, 16)
    out_ref[0] = jax.lax.bitcast_convert_type(packed, jnp.float32)


def _repack(table_t):
    out = pl.pallas_call(
        _repack_kernel,
        grid=(_N_BLK,),
        compiler_params=pltpu.CompilerParams(
            dimension_semantics=("arbitrary",)),
        in_specs=[pl.BlockSpec((EMB, _CBLK), lambda i: (0, i))],
        out_specs=pl.BlockSpec((1, _HROWS, ROW), lambda i: (i, 0, 0)),
        out_shape=jax.ShapeDtypeStruct((_N_BLK, _HROWS, ROW), jnp.float32),
    )(table_t)
    return out.reshape(_N_BLK * _HROWS, ROW)


def _make_sc_gather(num_rows):
    info = plsc.get_sparse_core_info()
    nw = info.num_cores * info.num_subcores
    b_per_w = num_rows // nw
    n_chunks = b_per_w // _CHUNK
    mesh = plsc.VectorSubcoreMesh(core_axis_name="c", subcore_axis_name="s")

    @functools.partial(
        pl.kernel,
        mesh=mesh,
        out_type=jax.ShapeDtypeStruct((num_rows, ROW), jnp.float32),
        scratch_types=[
            pltpu.VMEM((n_chunks, _CHUNK), jnp.int32),
            pltpu.VMEM((2, _CHUNK, ROW), jnp.float32),
            pltpu.SemaphoreType.DMA,
            pltpu.SemaphoreType.DMA,
        ],
    )
    def gather_kernel(uidx_hbm, wu_hbm, u_out, uidx_v, u_buf, gsem, wsem):
        wid = lax.axis_index("s") * info.num_cores + lax.axis_index("c")
        base = wid * b_per_w
        pltpu.sync_copy(uidx_hbm.at[pl.ds(wid * n_chunks, n_chunks)], uidx_v)
        writes = []
        for j in range(n_chunks):
            bb = j % 2
            if j >= 2:
                writes[j - 2].wait()
            cu = pltpu.async_copy(wu_hbm.at[uidx_v.at[j]], u_buf.at[bb], gsem)
            cu.wait()
            dst = pl.ds(base + j * _CHUNK, _CHUNK)
            writes.append(pltpu.async_copy(u_buf.at[bb], u_out.at[dst], wsem))
        for w in writes[-2:]:
            w.wait()

    return gather_kernel


def _unpack_half(x, half):
    bits = jax.lax.bitcast_convert_type(x, jnp.int32)
    hi = jax.lax.bitcast_convert_type(bits & jnp.int32(-65536), jnp.float32)
    lo = jax.lax.bitcast_convert_type(
        jax.lax.shift_left(bits, 16), jnp.float32)
    return jnp.where(half[:, None] == 0, hi, lo)


def _head_kernel(u_ref, v_ref, us_ref, vs_ref, uh_ref, vh_ref, g_ref,
                 lwu4_ref, lwv4_ref, linb_ref, selw4_ref, selb_ref, a_ref,
                 b_ref, t_ref, out_ref):
    u = _unpack_half(u_ref[...], uh_ref[...])
    v = _unpack_half(v_ref[...], vh_ref[...])
    lane_q = lax.broadcasted_iota(jnp.int32, (1, ROW), 1) % PACK
    u = jnp.where(lane_q == us_ref[...][:, None], u, 0.0)
    v = jnp.where(lane_q == vs_ref[...][:, None], v, 0.0)
    logit = (jnp.dot(u, lwu4_ref[...], preferred_element_type=jnp.float32)
             + jnp.dot(v, lwv4_ref[...], preferred_element_type=jnp.float32)
             + linb_ref[0, 0])
    s = (jnp.dot(u, selw4_ref[...], preferred_element_type=jnp.float32)
         + selb_ref[...])
    s = s - jnp.max(s, axis=1, keepdims=True)
    es = jnp.exp(s)
    sd = es / jnp.sum(es, axis=1, keepdims=True) + 1e-10
    t = (jnp.log(sd) + g_ref[...]) / t_ref[0, 0]
    t = t - jnp.max(t, axis=1, keepdims=True)
    et = jnp.exp(t)
    w = et / jnp.sum(et, axis=1, keepdims=True)
    eo = 1.0 / (1.0 + jnp.exp(-(logit * a_ref[...] + b_ref[...])))
    r = jnp.sum(eo * w, axis=1)
    out_ref[...] = jnp.clip(r, 0.0, 1.0)


def _run_head(u_emb, v_emb, u_sub, v_sub, u_half, v_half, g, lwu4, lwv4,
              lin_b, selw4, sel_b, a_prop, b_prop, t):
    n_blk = 8
    rows = B // n_blk
    full = lambda s: pl.BlockSpec(s, lambda i: (0,) * len(s))
    out = pl.pallas_call(
        _head_kernel,
        grid=(n_blk,),
        in_specs=[
            pl.BlockSpec((rows, ROW), lambda i: (i, 0)),
            pl.BlockSpec((rows, ROW), lambda i: (i, 0)),
            pl.BlockSpec((rows,), lambda i: (i,)),
            pl.BlockSpec((rows,), lambda i: (i,)),
            pl.BlockSpec((rows,), lambda i: (i,)),
            pl.BlockSpec((rows,), lambda i: (i,)),
            pl.BlockSpec((rows, E), lambda i: (i, 0)),
            full((ROW, 1)),
            full((ROW, 1)),
            full((1, 1)),
            full((ROW, E)),
            full((1, E)),
            full((1, E)),
            full((1, E)),
            full((1, 1)),
        ],
        out_specs=pl.BlockSpec((rows,), lambda i: (i,)),
        out_shape=jax.ShapeDtypeStruct((B,), jnp.float32),
    )(u_emb, v_emb, u_sub, v_sub, u_half, v_half, g, lwu4, lwv4,
      lin_b.reshape(1, 1), selw4, sel_b.reshape(1, E), a_prop.reshape(1, E),
      b_prop.reshape(1, E), t)
    return out


def kernel(x, T, W_user, H_item, lin_w, lin_b, sel_w, sel_b, a_prop, b_prop, g):
    user_idx = x[:, 0]
    item_idx = x[:, 1]
    uq = ((user_idx // _CBLK) * _HROWS
          + user_idx % _HROWS).reshape(B // _CHUNK, _CHUNK)
    iq = ((item_idx // _CBLK) * _HROWS
          + item_idx % _HROWS).reshape(B // _CHUNK, _CHUNK)
    usub = (user_idx // _QBLK) % PACK
    isub = (item_idx // _QBLK) % PACK
    uhalf = (user_idx // _HROWS) % 2
    ihalf = (item_idx // _HROWS) % 2
    gather = _make_sc_gather(B)
    w4 = _repack(W_user.T)
    u_emb = gather(uq, w4)
    h4 = _repack(H_item.T)
    v_emb = gather(iq, h4)
    lwu4 = jnp.repeat(lin_w[:EMB], PACK, axis=0)
    lwv4 = jnp.repeat(lin_w[EMB:], PACK, axis=0)
    selw4 = jnp.repeat(sel_w, PACK, axis=0)
    t = jnp.asarray(T, jnp.float32).reshape(1, 1)
    return _run_head(u_emb, v_emb, usub, isub, uhalf, ihalf, g, lwu4, lwv4,
                     lin_b, selw4, sel_b, a_prop, b_prop, t)


# speedup vs baseline: 1.2792x; 1.1623x over previous
import functools

import jax
import jax.numpy as jnp
from jax import lax
from jax.experimental import pallas as pl
from jax.experimental.pallas import tpu as pltpu
from jax.experimental.pallas import tpu_sc as plsc

B = 16384
EMB = 32
E = 8
PACK = 4
ROW = EMB * PACK

_CHUNK = 128
_CBLK = 32768
_QBLK = _CBLK // PACK
_N_USERS = 1000000
_N_BLK = -(-_N_USERS // _CBLK)
_PROWS = _N_BLK * _QBLK


_HROWS = _QBLK // 2


def _repack_kernel(in_ref, out_ref):
    blk = in_ref[...]
    t = jnp.transpose(jnp.reshape(blk, (ROW, _QBLK)), (1, 0))
    bi = jax.lax.bitcast_convert_type(t[:_HROWS], jnp.int32)
    bl = jax.lax.bitcast_convert_type(t[_HROWS:], jnp.int32)
    packed = (bi & jnp.int32(-65536)) | jax.lax.shift_right_logical(bl, 16)
    out_ref[0] = jax.lax.bitcast_convert_type(packed, jnp.float32)


def _repack(table_t):
    out = pl.pallas_call(
        _repack_kernel,
        grid=(_N_BLK,),
        compiler_params=pltpu.CompilerParams(
            dimension_semantics=("arbitrary",)),
        in_specs=[pl.BlockSpec((EMB, _CBLK), lambda i: (0, i))],
        out_specs=pl.BlockSpec((1, _HROWS, ROW), lambda i: (i, 0, 0)),
        out_shape=jax.ShapeDtypeStruct((_N_BLK, _HROWS, ROW), jnp.float32),
    )(table_t)
    return out.reshape(_N_BLK * _HROWS, ROW)


def _make_sc_gather(num_rows):
    info = plsc.get_sparse_core_info()
    nw = info.num_cores * info.num_subcores
    b_per_w = num_rows // nw
    n_chunks = b_per_w // _CHUNK
    mesh = plsc.VectorSubcoreMesh(core_axis_name="c", subcore_axis_name="s")

    @functools.partial(
        pl.kernel,
        mesh=mesh,
        out_type=jax.ShapeDtypeStruct((num_rows, ROW), jnp.float32),
        scratch_types=[
            pltpu.VMEM((n_chunks, _CHUNK), jnp.int32),
            pltpu.VMEM((2, _CHUNK, ROW), jnp.float32),
            pltpu.SemaphoreType.DMA,
            pltpu.SemaphoreType.DMA,
        ],
    )
    def gather_kernel(uidx_hbm, wu_hbm, u_out, uidx_v, u_buf, gsem, wsem):
        wid = lax.axis_index("s") * info.num_cores + lax.axis_index("c")
        base = wid * b_per_w
        pltpu.sync_copy(uidx_hbm.at[pl.ds(wid * n_chunks, n_chunks)], uidx_v)
        writes = []
        for j in range(n_chunks):
            bb = j % 2
            if j >= 2:
                writes[j - 2].wait()
            cu = pltpu.async_copy(wu_hbm.at[uidx_v.at[j]], u_buf.at[bb], gsem)
            cu.wait()
            dst = pl.ds(base + j * _CHUNK, _CHUNK)
            writes.append(pltpu.async_copy(u_buf.at[bb], u_out.at[dst], wsem))
        for w in writes[-2:]:
            w.wait()

    return gather_kernel


def _unpack_half(x, half):
    bits = jax.lax.bitcast_convert_type(x, jnp.int32)
    hi = jax.lax.bitcast_convert_type(bits & jnp.int32(-65536), jnp.float32)
    lo = jax.lax.bitcast_convert_type(
        jax.lax.shift_left(bits, 16), jnp.float32)
    return jnp.where(half[:, None] == 0, hi, lo)


def _head_kernel(u_ref, v_ref, us_ref, vs_ref, uh_ref, vh_ref, g_ref,
                 lwu4_ref, lwv4_ref, linb_ref, selw4_ref, selb_ref, a_ref,
                 b_ref, t_ref, out_ref):
    u = _unpack_half(u_ref[...], uh_ref[...])
    v = _unpack_half(v_ref[...], vh_ref[...])
    lane_q = lax.broadcasted_iota(jnp.int32, (1, ROW), 1) % PACK
    u = jnp.where(lane_q == us_ref[...][:, None], u, 0.0)
    v = jnp.where(lane_q == vs_ref[...][:, None], v, 0.0)
    logit = (jnp.dot(u, lwu4_ref[...], preferred_element_type=jnp.float32)
             + jnp.dot(v, lwv4_ref[...], preferred_element_type=jnp.float32)
             + linb_ref[0, 0])
    s = (jnp.dot(u, selw4_ref[...], preferred_element_type=jnp.float32)
         + selb_ref[...])
    s = s - jnp.max(s, axis=1, keepdims=True)
    es = jnp.exp(s)
    sd = es / jnp.sum(es, axis=1, keepdims=True) + 1e-10
    t = (jnp.log(sd) + g_ref[...]) / t_ref[0, 0]
    t = t - jnp.max(t, axis=1, keepdims=True)
    et = jnp.exp(t)
    w = et / jnp.sum(et, axis=1, keepdims=True)
    eo = 1.0 / (1.0 + jnp.exp(-(logit * a_ref[...] + b_ref[...])))
    r = jnp.sum(eo * w, axis=1)
    out_ref[...] = jnp.clip(r, 0.0, 1.0)


def _run_head(u_emb, v_emb, u_sub, v_sub, u_half, v_half, g, lwu4, lwv4,
              lin_b, selw4, sel_b, a_prop, b_prop, t):
    n_blk = 8
    rows = B // n_blk
    full = lambda s: pl.BlockSpec(s, lambda i: (0,) * len(s))
    out = pl.pallas_call(
        _head_kernel,
        grid=(n_blk,),
        in_specs=[
            pl.BlockSpec((rows, ROW), lambda i: (i, 0)),
            pl.BlockSpec((rows, ROW), lambda i: (i, 0)),
            pl.BlockSpec((rows,), lambda i: (i,)),
            pl.BlockSpec((rows,), lambda i: (i,)),
            pl.BlockSpec((rows,), lambda i: (i,)),
            pl.BlockSpec((rows,), lambda i: (i,)),
            pl.BlockSpec((rows, E), lambda i: (i, 0)),
            full((ROW, 1)),
            full((ROW, 1)),
            full((1, 1)),
            full((ROW, E)),
            full((1, E)),
            full((1, E)),
            full((1, E)),
            full((1, 1)),
        ],
        out_specs=pl.BlockSpec((rows,), lambda i: (i,)),
        out_shape=jax.ShapeDtypeStruct((B,), jnp.float32),
    )(u_emb, v_emb, u_sub, v_sub, u_half, v_half, g, lwu4, lwv4,
      lin_b.reshape(1, 1), selw4, sel_b.reshape(1, E), a_prop.reshape(1, E),
      b_prop.reshape(1, E), t)
    return out


def kernel(x, T, W_user, H_item, lin_w, lin_b, sel_w, sel_b, a_prop, b_prop, g):
    user_idx = x[:, 0]
    item_idx = x[:, 1]
    uq = ((user_idx // _CBLK) * _HROWS
          + user_idx % _HROWS).reshape(B // _CHUNK, _CHUNK)
    iq = ((item_idx // _CBLK) * _HROWS
          + item_idx % _HROWS).reshape(B // _CHUNK, _CHUNK)
    usub = (user_idx // _QBLK) % PACK
    isub = (item_idx // _QBLK) % PACK
    uhalf = (user_idx // _HROWS) % 2
    ihalf = (item_idx // _HROWS) % 2
    gather = _make_sc_gather(B)
    w4 = _repack(W_user.T)
    u_emb = gather(uq, w4)
    h4 = _repack(H_item.T)
    v_emb = gather(iq, h4)
    lwu4 = jnp.repeat(lin_w[:EMB], PACK, axis=0)
    lwv4 = jnp.repeat(lin_w[EMB:], PACK, axis=0)
    selw4 = jnp.repeat(sel_w, PACK, axis=0)
    t = jnp.asarray(T, jnp.float32).reshape(1, 1)
    return _run_head(u_emb, v_emb, usub, isub, uhalf, ihalf, g, lwu4, lwv4,
                     lin_b, selw4, sel_b, a_prop, b_prop, t)


# speedup vs baseline: 1.3169x; 1.0295x over previous
import functools

import jax
import jax.numpy as jnp
from jax import lax
from jax.experimental import pallas as pl
from jax.experimental.pallas import tpu as pltpu
from jax.experimental.pallas import tpu_sc as plsc

B = 16384
EMB = 32
E = 8
PACK = 4
ROW = EMB * PACK

_CHUNK = 128
_CBLK = 65536
_QBLK = _CBLK // PACK
_N_USERS = 1000000
_N_BLK = -(-_N_USERS // _CBLK)
_PROWS = _N_BLK * _QBLK


_HROWS = _QBLK // 2


def _repack_kernel(in_ref, out_ref):
    blk = in_ref[...]
    t = jnp.transpose(jnp.reshape(blk, (ROW, _QBLK)), (1, 0))
    bi = jax.lax.bitcast_convert_type(t[:_HROWS], jnp.int32)
    bl = jax.lax.bitcast_convert_type(t[_HROWS:], jnp.int32)
    packed = (bi & jnp.int32(-65536)) | jax.lax.shift_right_logical(bl, 16)
    out_ref[0] = jax.lax.bitcast_convert_type(packed, jnp.float32)


def _repack(table_t):
    out = pl.pallas_call(
        _repack_kernel,
        grid=(_N_BLK,),
        compiler_params=pltpu.CompilerParams(
            dimension_semantics=("arbitrary",)),
        in_specs=[pl.BlockSpec((EMB, _CBLK), lambda i: (0, i))],
        out_specs=pl.BlockSpec((1, _HROWS, ROW), lambda i: (i, 0, 0)),
        out_shape=jax.ShapeDtypeStruct((_N_BLK, _HROWS, ROW), jnp.float32),
    )(table_t)
    return out.reshape(_N_BLK * _HROWS, ROW)


def _make_sc_gather(num_rows):
    info = plsc.get_sparse_core_info()
    nw = info.num_cores * info.num_subcores
    b_per_w = num_rows // nw
    n_chunks = b_per_w // _CHUNK
    mesh = plsc.VectorSubcoreMesh(core_axis_name="c", subcore_axis_name="s")

    @functools.partial(
        pl.kernel,
        mesh=mesh,
        out_type=jax.ShapeDtypeStruct((num_rows, ROW), jnp.float32),
        scratch_types=[
            pltpu.VMEM((n_chunks, _CHUNK), jnp.int32),
            pltpu.VMEM((2, _CHUNK, ROW), jnp.float32),
            pltpu.SemaphoreType.DMA,
            pltpu.SemaphoreType.DMA,
        ],
    )
    def gather_kernel(uidx_hbm, wu_hbm, u_out, uidx_v, u_buf, gsem, wsem):
        wid = lax.axis_index("s") * info.num_cores + lax.axis_index("c")
        base = wid * b_per_w
        pltpu.sync_copy(uidx_hbm.at[pl.ds(wid * n_chunks, n_chunks)], uidx_v)
        writes = []
        for j in range(n_chunks):
            bb = j % 2
            if j >= 2:
                writes[j - 2].wait()
            cu = pltpu.async_copy(wu_hbm.at[uidx_v.at[j]], u_buf.at[bb], gsem)
            cu.wait()
            dst = pl.ds(base + j * _CHUNK, _CHUNK)
            writes.append(pltpu.async_copy(u_buf.at[bb], u_out.at[dst], wsem))
        for w in writes[-2:]:
            w.wait()

    return gather_kernel


def _unpack_half(x, half):
    bits = jax.lax.bitcast_convert_type(x, jnp.int32)
    hi = jax.lax.bitcast_convert_type(bits & jnp.int32(-65536), jnp.float32)
    lo = jax.lax.bitcast_convert_type(
        jax.lax.shift_left(bits, 16), jnp.float32)
    return jnp.where(half[:, None] == 0, hi, lo)


def _head_kernel(u_ref, v_ref, us_ref, vs_ref, uh_ref, vh_ref, g_ref,
                 lwu4_ref, lwv4_ref, linb_ref, selw4_ref, selb_ref, a_ref,
                 b_ref, t_ref, out_ref):
    u = _unpack_half(u_ref[...], uh_ref[...])
    v = _unpack_half(v_ref[...], vh_ref[...])
    lane_q = lax.broadcasted_iota(jnp.int32, (1, ROW), 1) % PACK
    u = jnp.where(lane_q == us_ref[...][:, None], u, 0.0)
    v = jnp.where(lane_q == vs_ref[...][:, None], v, 0.0)
    logit = (jnp.dot(u, lwu4_ref[...], preferred_element_type=jnp.float32)
             + jnp.dot(v, lwv4_ref[...], preferred_element_type=jnp.float32)
             + linb_ref[0, 0])
    s = (jnp.dot(u, selw4_ref[...], preferred_element_type=jnp.float32)
         + selb_ref[...])
    s = s - jnp.max(s, axis=1, keepdims=True)
    es = jnp.exp(s)
    sd = es / jnp.sum(es, axis=1, keepdims=True) + 1e-10
    t = (jnp.log(sd) + g_ref[...]) / t_ref[0, 0]
    t = t - jnp.max(t, axis=1, keepdims=True)
    et = jnp.exp(t)
    w = et / jnp.sum(et, axis=1, keepdims=True)
    eo = 1.0 / (1.0 + jnp.exp(-(logit * a_ref[...] + b_ref[...])))
    r = jnp.sum(eo * w, axis=1)
    out_ref[...] = jnp.clip(r, 0.0, 1.0)


def _run_head(u_emb, v_emb, u_sub, v_sub, u_half, v_half, g, lwu4, lwv4,
              lin_b, selw4, sel_b, a_prop, b_prop, t):
    n_blk = 8
    rows = B // n_blk
    full = lambda s: pl.BlockSpec(s, lambda i: (0,) * len(s))
    out = pl.pallas_call(
        _head_kernel,
        grid=(n_blk,),
        in_specs=[
            pl.BlockSpec((rows, ROW), lambda i: (i, 0)),
            pl.BlockSpec((rows, ROW), lambda i: (i, 0)),
            pl.BlockSpec((rows,), lambda i: (i,)),
            pl.BlockSpec((rows,), lambda i: (i,)),
            pl.BlockSpec((rows,), lambda i: (i,)),
            pl.BlockSpec((rows,), lambda i: (i,)),
            pl.BlockSpec((rows, E), lambda i: (i, 0)),
            full((ROW, 1)),
            full((ROW, 1)),
            full((1, 1)),
            full((ROW, E)),
            full((1, E)),
            full((1, E)),
            full((1, E)),
            full((1, 1)),
        ],
        out_specs=pl.BlockSpec((rows,), lambda i: (i,)),
        out_shape=jax.ShapeDtypeStruct((B,), jnp.float32),
    )(u_emb, v_emb, u_sub, v_sub, u_half, v_half, g, lwu4, lwv4,
      lin_b.reshape(1, 1), selw4, sel_b.reshape(1, E), a_prop.reshape(1, E),
      b_prop.reshape(1, E), t)
    return out


def kernel(x, T, W_user, H_item, lin_w, lin_b, sel_w, sel_b, a_prop, b_prop, g):
    user_idx = x[:, 0]
    item_idx = x[:, 1]
    uq = ((user_idx // _CBLK) * _HROWS
          + user_idx % _HROWS).reshape(B // _CHUNK, _CHUNK)
    iq = ((item_idx // _CBLK) * _HROWS
          + item_idx % _HROWS).reshape(B // _CHUNK, _CHUNK)
    usub = (user_idx // _QBLK) % PACK
    isub = (item_idx // _QBLK) % PACK
    uhalf = (user_idx // _HROWS) % 2
    ihalf = (item_idx // _HROWS) % 2
    gather = _make_sc_gather(B)
    w4 = _repack(W_user.T)
    u_emb = gather(uq, w4)
    h4 = _repack(H_item.T)
    v_emb = gather(iq, h4)
    lwu4 = jnp.repeat(lin_w[:EMB], PACK, axis=0)
    lwv4 = jnp.repeat(lin_w[EMB:], PACK, axis=0)
    selw4 = jnp.repeat(sel_w, PACK, axis=0)
    t = jnp.asarray(T, jnp.float32).reshape(1, 1)
    return _run_head(u_emb, v_emb, usub, isub, uhalf, ihalf, g, lwu4, lwv4,
                     lin_b, selw4, sel_b, a_prop, b_prop, t)


# speedup vs baseline: 1.3353x; 1.0140x over previous
import functools

import jax
import jax.numpy as jnp
from jax import lax
from jax.experimental import pallas as pl
from jax.experimental.pallas import tpu as pltpu
from jax.experimental.pallas import tpu_sc as plsc

B = 16384
EMB = 32
E = 8
PACK = 4
ROW = EMB * PACK

_CHUNK = 128
_CBLK = 131072
_QBLK = _CBLK // PACK
_N_USERS = 1000000
_N_BLK = -(-_N_USERS // _CBLK)
_PROWS = _N_BLK * _QBLK


_HROWS = _QBLK // 2


def _repack_kernel(in_ref, out_ref):
    blk = in_ref[...]
    t = jnp.transpose(jnp.reshape(blk, (ROW, _QBLK)), (1, 0))
    bi = jax.lax.bitcast_convert_type(t[:_HROWS], jnp.int32)
    bl = jax.lax.bitcast_convert_type(t[_HROWS:], jnp.int32)
    packed = (bi & jnp.int32(-65536)) | jax.lax.shift_right_logical(bl, 16)
    out_ref[0] = jax.lax.bitcast_convert_type(packed, jnp.float32)


def _repack(table_t):
    out = pl.pallas_call(
        _repack_kernel,
        grid=(_N_BLK,),
        compiler_params=pltpu.CompilerParams(
            dimension_semantics=("arbitrary",)),
        in_specs=[pl.BlockSpec((EMB, _CBLK), lambda i: (0, i))],
        out_specs=pl.BlockSpec((1, _HROWS, ROW), lambda i: (i, 0, 0)),
        out_shape=jax.ShapeDtypeStruct((_N_BLK, _HROWS, ROW), jnp.float32),
    )(table_t)
    return out.reshape(_N_BLK * _HROWS, ROW)


def _make_sc_gather(num_rows):
    info = plsc.get_sparse_core_info()
    nw = info.num_cores * info.num_subcores
    b_per_w = num_rows // nw
    n_chunks = b_per_w // _CHUNK
    mesh = plsc.VectorSubcoreMesh(core_axis_name="c", subcore_axis_name="s")

    @functools.partial(
        pl.kernel,
        mesh=mesh,
        out_type=jax.ShapeDtypeStruct((num_rows, ROW), jnp.float32),
        scratch_types=[
            pltpu.VMEM((n_chunks, _CHUNK), jnp.int32),
            pltpu.VMEM((2, _CHUNK, ROW), jnp.float32),
            pltpu.SemaphoreType.DMA,
            pltpu.SemaphoreType.DMA,
        ],
    )
    def gather_kernel(uidx_hbm, wu_hbm, u_out, uidx_v, u_buf, gsem, wsem):
        wid = lax.axis_index("s") * info.num_cores + lax.axis_index("c")
        base = wid * b_per_w
        pltpu.sync_copy(uidx_hbm.at[pl.ds(wid * n_chunks, n_chunks)], uidx_v)
        writes = []
        for j in range(n_chunks):
            bb = j % 2
            if j >= 2:
                writes[j - 2].wait()
            cu = pltpu.async_copy(wu_hbm.at[uidx_v.at[j]], u_buf.at[bb], gsem)
            cu.wait()
            dst = pl.ds(base + j * _CHUNK, _CHUNK)
            writes.append(pltpu.async_copy(u_buf.at[bb], u_out.at[dst], wsem))
        for w in writes[-2:]:
            w.wait()

    return gather_kernel


def _unpack_half(x, half):
    bits = jax.lax.bitcast_convert_type(x, jnp.int32)
    hi = jax.lax.bitcast_convert_type(bits & jnp.int32(-65536), jnp.float32)
    lo = jax.lax.bitcast_convert_type(
        jax.lax.shift_left(bits, 16), jnp.float32)
    return jnp.where(half[:, None] == 0, hi, lo)


def _head_kernel(u_ref, v_ref, us_ref, vs_ref, uh_ref, vh_ref, g_ref,
                 lwu4_ref, lwv4_ref, linb_ref, selw4_ref, selb_ref, a_ref,
                 b_ref, t_ref, out_ref):
    u = _unpack_half(u_ref[...], uh_ref[...])
    v = _unpack_half(v_ref[...], vh_ref[...])
    lane_q = lax.broadcasted_iota(jnp.int32, (1, ROW), 1) % PACK
    u = jnp.where(lane_q == us_ref[...][:, None], u, 0.0)
    v = jnp.where(lane_q == vs_ref[...][:, None], v, 0.0)
    logit = (jnp.dot(u, lwu4_ref[...], preferred_element_type=jnp.float32)
             + jnp.dot(v, lwv4_ref[...], preferred_element_type=jnp.float32)
             + linb_ref[0, 0])
    s = (jnp.dot(u, selw4_ref[...], preferred_element_type=jnp.float32)
         + selb_ref[...])
    s = s - jnp.max(s, axis=1, keepdims=True)
    es = jnp.exp(s)
    sd = es / jnp.sum(es, axis=1, keepdims=True) + 1e-10
    t = (jnp.log(sd) + g_ref[...]) / t_ref[0, 0]
    t = t - jnp.max(t, axis=1, keepdims=True)
    et = jnp.exp(t)
    w = et / jnp.sum(et, axis=1, keepdims=True)
    eo = 1.0 / (1.0 + jnp.exp(-(logit * a_ref[...] + b_ref[...])))
    r = jnp.sum(eo * w, axis=1)
    out_ref[...] = jnp.clip(r, 0.0, 1.0)


def _run_head(u_emb, v_emb, u_sub, v_sub, u_half, v_half, g, lwu4, lwv4,
              lin_b, selw4, sel_b, a_prop, b_prop, t):
    n_blk = 8
    rows = B // n_blk
    full = lambda s: pl.BlockSpec(s, lambda i: (0,) * len(s))
    out = pl.pallas_call(
        _head_kernel,
        grid=(n_blk,),
        in_specs=[
            pl.BlockSpec((rows, ROW), lambda i: (i, 0)),
            pl.BlockSpec((rows, ROW), lambda i: (i, 0)),
            pl.BlockSpec((rows,), lambda i: (i,)),
            pl.BlockSpec((rows,), lambda i: (i,)),
            pl.BlockSpec((rows,), lambda i: (i,)),
            pl.BlockSpec((rows,), lambda i: (i,)),
            pl.BlockSpec((rows, E), lambda i: (i, 0)),
            full((ROW, 1)),
            full((ROW, 1)),
            full((1, 1)),
            full((ROW, E)),
            full((1, E)),
            full((1, E)),
            full((1, E)),
            full((1, 1)),
        ],
        out_specs=pl.BlockSpec((rows,), lambda i: (i,)),
        out_shape=jax.ShapeDtypeStruct((B,), jnp.float32),
    )(u_emb, v_emb, u_sub, v_sub, u_half, v_half, g, lwu4, lwv4,
      lin_b.reshape(1, 1), selw4, sel_b.reshape(1, E), a_prop.reshape(1, E),
      b_prop.reshape(1, E), t)
    return out


def kernel(x, T, W_user, H_item, lin_w, lin_b, sel_w, sel_b, a_prop, b_prop, g):
    user_idx = x[:, 0]
    item_idx = x[:, 1]
    uq = ((user_idx // _CBLK) * _HROWS
          + user_idx % _HROWS).reshape(B // _CHUNK, _CHUNK)
    iq = ((item_idx // _CBLK) * _HROWS
          + item_idx % _HROWS).reshape(B // _CHUNK, _CHUNK)
    usub = (user_idx // _QBLK) % PACK
    isub = (item_idx // _QBLK) % PACK
    uhalf = (user_idx // _HROWS) % 2
    ihalf = (item_idx // _HROWS) % 2
    gather = _make_sc_gather(B)
    w4 = _repack(W_user.T)
    u_emb = gather(uq, w4)
    h4 = _repack(H_item.T)
    v_emb = gather(iq, h4)
    lwu4 = jnp.repeat(lin_w[:EMB], PACK, axis=0)
    lwv4 = jnp.repeat(lin_w[EMB:], PACK, axis=0)
    selw4 = jnp.repeat(sel_w, PACK, axis=0)
    t = jnp.asarray(T, jnp.float32).reshape(1, 1)
    return _run_head(u_emb, v_emb, usub, isub, uhalf, ihalf, g, lwu4, lwv4,
                     lin_b, selw4, sel_b, a_prop, b_prop, t)
